# Initial kernel scaffold; baseline (speedup 1.0000x reference)
import jax
import jax.numpy as jnp
from jax.experimental import pallas as pl


def kernel(embeddings, labels):
    raise NotImplementedError("write your pallas kernel here")


# speedup vs baseline: 6.9952x; 6.9952x over previous
import jax
import jax.numpy as jnp
from jax.experimental import pallas as pl
from jax.experimental.pallas import tpu as pltpu

B = 16384
D = 64
C = 1000
MARGIN = 0.3
EPS = 1e-12
BLK = 2048
NBLK = B // BLK


def _loss_body(emb_ref, lab_col_ref, lab_row_ref, out_ref,
               en_ref, sums_ref, counts_ref, w_ref, vsum_ref):
    e = emb_ref[...]
    ss = jnp.sum(e * e, axis=1, keepdims=True)
    nrm = jnp.maximum(jnp.sqrt(ss), EPS)
    en_ref[...] = e / nrm

    for k in range(NBLK):
        en_b = en_ref[pl.ds(k * BLK, BLK), :]
        lab_r = lab_row_ref[:, pl.ds(k * BLK, BLK)]
        iota_c = jax.lax.broadcasted_iota(jnp.int32, (C, BLK), 0)
        oh_t = (iota_c == lab_r).astype(jnp.float32)
        ps = jax.lax.dot_general(oh_t, en_b, (((1,), (0,)), ((), ())),
                                 preferred_element_type=jnp.float32)
        pc = jnp.sum(oh_t, axis=1, keepdims=True)
        if k == 0:
            sums_ref[...] = ps
            counts_ref[...] = pc
        else:
            sums_ref[...] += ps
            counts_ref[...] += pc

    counts = counts_ref[...]
    safe = jnp.maximum(counts, 1.0)
    cen = sums_ref[...] / safe
    cn = jnp.maximum(jnp.sqrt(jnp.sum(cen * cen, axis=1, keepdims=True)), EPS)
    cen = cen / cn

    cen2 = cen * cen
    sq_col = jnp.sum(cen2, axis=1, keepdims=True)
    ones_row = jnp.ones((1, D), jnp.float32)
    sq_row = jax.lax.dot_general(ones_row, cen2, (((1,), (1,)), ((), ())),
                                 preferred_element_type=jnp.float32)
    g = jax.lax.dot_general(cen, cen, (((1,), (1,)), ((), ())),
                            preferred_element_type=jnp.float32)
    d2 = jnp.maximum(sq_col + sq_row - 2.0 * g, 0.0)
    dist = jnp.sqrt(d2)
    row_i = jax.lax.broadcasted_iota(jnp.int32, (C, C), 0)
    col_i = jax.lax.broadcasted_iota(jnp.int32, (C, C), 1)
    dist = jnp.where(row_i == col_i, jnp.inf, dist)
    minv = jnp.min(dist, axis=1, keepdims=True)
    nearest = jnp.min(jnp.where(dist == minv, col_i, jnp.int32(2 ** 30)),
                      axis=1, keepdims=True)

    oh_n = (col_i == nearest).astype(jnp.float32)
    c_neg = jax.lax.dot_general(oh_n, cen, (((1,), (0,)), ((), ())),
                                preferred_element_type=jnp.float32)
    w_ref[...] = c_neg - cen

    for k in range(NBLK):
        en_b = en_ref[pl.ds(k * BLK, BLK), :]
        lab_c = lab_col_ref[pl.ds(k * BLK, BLK), :]
        lab_r = lab_row_ref[:, pl.ds(k * BLK, BLK)]
        iota_b = jax.lax.broadcasted_iota(jnp.int32, (BLK, C), 1)
        oh = (iota_b == lab_c).astype(jnp.float32)
        iota_c = jax.lax.broadcasted_iota(jnp.int32, (C, BLK), 0)
        oh_t = (iota_c == lab_r).astype(jnp.float32)
        w_rows = jax.lax.dot_general(oh, w_ref[...], (((1,), (0,)), ((), ())),
                                     preferred_element_type=jnp.float32)
        t = jnp.sum(en_b * w_rows, axis=1, keepdims=True)
        vals = jnp.maximum(t + MARGIN, 0.0)
        pv = jax.lax.dot_general(oh_t, vals, (((1,), (0,)), ((), ())),
                                 preferred_element_type=jnp.float32)
        if k == 0:
            vsum_ref[...] = pv
        else:
            vsum_ref[...] += pv

    per_class = vsum_ref[...] / safe
    present = (counts > 0.0).astype(jnp.float32)
    num = jnp.sum(per_class * present, axis=0, keepdims=True)
    den = jnp.maximum(jnp.sum(present, axis=0, keepdims=True), 1.0)
    out_ref[...] = num / den


def kernel(embeddings, labels):
    lab_col = labels.reshape(B, 1)
    lab_row = labels.reshape(1, B)
    out = pl.pallas_call(
        _loss_body,
        out_shape=jax.ShapeDtypeStruct((1, 1), jnp.float32),
        scratch_shapes=[
            pltpu.VMEM((B, D), jnp.float32),
            pltpu.VMEM((C, D), jnp.float32),
            pltpu.VMEM((C, 1), jnp.float32),
            pltpu.VMEM((C, D), jnp.float32),
            pltpu.VMEM((C, 1), jnp.float32),
        ],
    )(embeddings, lab_col, lab_row)
    return out[0, 0]
